# baseline (speedup 1.0000x reference)
import functools

import jax
import jax.numpy as jnp
from jax import lax
from jax.experimental import pallas as pl
from jax.experimental.pallas import tpu as pltpu
from jax.experimental.pallas import tpu_sc as plsc

_ROW = 3
_D = 128


@functools.partial(
    pl.kernel,
    out_type=jax.ShapeDtypeStruct((_D,), jnp.float32),
    mesh=plsc.VectorSubcoreMesh(core_axis_name="c", subcore_axis_name="s"),
    scratch_types=[pltpu.VMEM((_D,), jnp.float32)],
)
def _gather_row(table_hbm, out_hbm, row_v):
    cid = lax.axis_index("c")
    sid = lax.axis_index("s")

    @pl.when(jnp.logical_and(cid == 0, sid == 0))
    def _():
        pltpu.sync_copy(table_hbm.at[_ROW], row_v)
        pltpu.sync_copy(row_v, out_hbm)


def kernel(inputs):
    return _gather_row(inputs)


# speedup vs baseline: 1.0970x; 1.0970x over previous
import functools

import jax
import jax.numpy as jnp
from jax import lax
from jax.experimental import pallas as pl
from jax.experimental.pallas import tpu as pltpu
from jax.experimental.pallas import tpu_sc as plsc

_ROW = 3
_D = 128


@functools.partial(
    pl.kernel,
    out_type=jax.ShapeDtypeStruct((_D,), jnp.float32),
    mesh=plsc.ScalarSubcoreMesh(axis_name="c"),
)
def _gather_row(table_hbm, out_hbm):
    @pl.when(lax.axis_index("c") == 0)
    def _():
        pltpu.sync_copy(table_hbm.at[_ROW], out_hbm)


def kernel(inputs):
    return _gather_row(inputs)


# speedup vs baseline: 20.6801x; 18.8508x over previous
import jax
import jax.numpy as jnp
from jax.experimental import pallas as pl
from jax.experimental.pallas import tpu as pltpu

_ROW = 3
_D = 128


def _copy_row(in_ref, out_ref, sem):
    pltpu.make_async_copy(in_ref.at[_ROW], out_ref, sem).start()
    pltpu.make_async_copy(in_ref.at[_ROW], out_ref, sem).wait()


def kernel(inputs):
    return pl.pallas_call(
        _copy_row,
        out_shape=jax.ShapeDtypeStruct((_D,), jnp.float32),
        in_specs=[pl.BlockSpec(memory_space=pl.ANY)],
        out_specs=pl.BlockSpec(memory_space=pl.ANY),
        scratch_shapes=[pltpu.SemaphoreType.DMA],
    )(inputs)
